# Initial kernel scaffold; baseline (speedup 1.0000x reference)
import jax
import jax.numpy as jnp
from jax.experimental import pallas as pl


def kernel(x, adj, W, b):
    raise NotImplementedError("write your pallas kernel here")


# speedup vs baseline: 5.4133x; 5.4133x over previous
import functools

import jax
import jax.numpy as jnp
from jax import lax
from jax.experimental import pallas as pl
from jax.experimental.pallas import tpu as pltpu
from jax.experimental.pallas import tpu_sc as plsc

N_NODES = 10000
N_EDGES = 320000
D_IN = 128
D_OUT = 128

NC = 2
NS = 16

N_PAD = 10240
ROWS_PER_TILE = N_PAD // NS

EDGES_PER_TILE = N_EDGES // (NC * NS)
CHUNK = 80
NCHUNKS = EDGES_PER_TILE // CHUNK
assert NCHUNKS * CHUNK == EDGES_PER_TILE

MM_BLOCK = 1000
CB_BLOCK = 1000


def _matmul_body(x_ref, w_ref, out_ref):
    out_ref[...] = jnp.dot(x_ref[...], w_ref[...],
                           preferred_element_type=jnp.float32)


def _matmul(x, W):
    return pl.pallas_call(
        _matmul_body,
        grid=(N_NODES // MM_BLOCK,),
        in_specs=[
            pl.BlockSpec((MM_BLOCK, D_IN), lambda i: (i, 0)),
            pl.BlockSpec((D_IN, D_OUT), lambda i: (0, 0)),
        ],
        out_specs=pl.BlockSpec((MM_BLOCK, D_OUT), lambda i: (i, 0)),
        out_shape=jax.ShapeDtypeStruct((N_NODES, D_OUT), jnp.float32),
    )(x, W)


def _sc_body(h_hbm, src_hbm, dst_hbm, part_hbm,
             sidx_v, didx_v, rows_v, acc_sh, sem):
    cid = lax.axis_index("c")
    sid = lax.axis_index("s")
    row0 = sid * ROWS_PER_TILE

    zvec = jnp.zeros((16,), jnp.float32)

    def _zfill(t, _):
        r = t // (D_OUT // 16)
        j = t % (D_OUT // 16)
        rows_v[r, pl.ds(j * 16, 16)] = zvec
        return 0

    lax.fori_loop(0, CHUNK * (D_OUT // 16), _zfill, 0)
    for k in range(ROWS_PER_TILE // CHUNK):
        pltpu.sync_copy(rows_v, acc_sh.at[pl.ds(row0 + k * CHUNK, CHUNK)])
    plsc.subcore_barrier()

    ebase = (cid * NS + sid) * EDGES_PER_TILE

    def _chunk(g, _):
        off = ebase + g * CHUNK
        pltpu.sync_copy(src_hbm.at[pl.ds(off, CHUNK)], sidx_v)
        pltpu.sync_copy(dst_hbm.at[pl.ds(off, CHUNK)], didx_v)
        pltpu.async_copy(h_hbm.at[sidx_v], rows_v, sem).wait()
        pltpu.sync_copy(rows_v, acc_sh.at[didx_v], add=True)
        return 0

    lax.fori_loop(0, NCHUNKS, _chunk, 0)
    plsc.subcore_barrier()

    pltpu.sync_copy(
        acc_sh.at[pl.ds(row0, ROWS_PER_TILE)],
        part_hbm.at[cid, pl.ds(row0, ROWS_PER_TILE)],
    )


_sc_aggregate = functools.partial(
    pl.kernel,
    out_type=jax.ShapeDtypeStruct((NC, N_PAD, D_OUT), jnp.float32),
    mesh=plsc.VectorSubcoreMesh(core_axis_name="c", subcore_axis_name="s"),
    scratch_types=[
        pltpu.VMEM((CHUNK,), jnp.int32),
        pltpu.VMEM((CHUNK,), jnp.int32),
        pltpu.VMEM((CHUNK, D_OUT), jnp.float32),
        pltpu.VMEM_SHARED((N_PAD, D_OUT), jnp.float32),
        pltpu.SemaphoreType.DMA,
    ],
)(_sc_body)


def _combine_body(p_ref, b_ref, out_ref):
    out_ref[...] = p_ref[0] + p_ref[1] + b_ref[...]


def _combine(partials, b2d):
    return pl.pallas_call(
        _combine_body,
        grid=(N_NODES // CB_BLOCK,),
        in_specs=[
            pl.BlockSpec((NC, CB_BLOCK, D_OUT), lambda i: (0, i, 0)),
            pl.BlockSpec((1, D_OUT), lambda i: (0, 0)),
        ],
        out_specs=pl.BlockSpec((CB_BLOCK, D_OUT), lambda i: (i, 0)),
        out_shape=jax.ShapeDtypeStruct((N_NODES, D_OUT), jnp.float32),
    )(partials, b2d)


def kernel(x, adj, W, b):
    h = _matmul(x, W)
    partials = _sc_aggregate(h, adj[0], adj[1])
    return _combine(partials, b.reshape(1, D_OUT))
